# baseline (speedup 1.0000x reference)
import functools

import jax
import jax.numpy as jnp
from jax import lax
from jax.experimental import pallas as pl
from jax.experimental.pallas import tpu as pltpu
from jax.experimental.pallas import tpu_sc as plsc


def kernel(row_embed, col_embed):
    H, D2 = row_embed.shape
    W = col_embed.shape[0]

    NW = 32
    RPW = H // NW
    L = 16
    NV = D2 // L
    BR = 16

    mesh = plsc.VectorSubcoreMesh(core_axis_name="c", subcore_axis_name="s")

    @functools.partial(
        pl.kernel,
        mesh=mesh,
        out_type=jax.ShapeDtypeStruct((H * W, 2 * D2), jnp.float32),
        scratch_types=[
            pltpu.VMEM((RPW, D2), jnp.float32),
            pltpu.VMEM((BR, D2), jnp.float32),
            pltpu.VMEM((BR, D2), jnp.float32),
            pltpu.SemaphoreType.DMA,
        ],
    )
    def row_half_kernel(row_hbm, out_hbm, myrows_v, b0, b1, wsem):
        wid = lax.axis_index("s") * 2 + lax.axis_index("c")
        base_i = wid * RPW
        pltpu.sync_copy(row_hbm.at[pl.ds(base_i, RPW)], myrows_v)
        writes = []
        bufs = (b0, b1)
        for t in range(RPW):
            bcast = bufs[t]
            vals = [myrows_v[t, pl.ds(v * L, L)] for v in range(NV)]
            for r in range(BR):
                for v in range(NV):
                    bcast[r, pl.ds(v * L, L)] = vals[v]
            row0 = (base_i + t) * W
            for q in range(W // BR):
                writes.append(
                    pltpu.async_copy(
                        bcast,
                        out_hbm.at[pl.ds(row0 + q * BR, BR), pl.ds(0, D2)],
                        wsem,
                    )
                )
        for wcp in writes:
            wcp.wait()

    sc_out = row_half_kernel(row_embed)

    BLK = 16

    def col_half_body(col_ref, _, out_ref):
        for k in range(BLK):
            out_ref[k * W:(k + 1) * W, :] = col_ref[...]

    out = pl.pallas_call(
        col_half_body,
        grid=(H // BLK,),
        in_specs=[
            pl.BlockSpec((W, D2), lambda i: (0, 0)),
            pl.BlockSpec(memory_space=pl.ANY),
        ],
        out_specs=pl.BlockSpec((BLK * W, D2), lambda i: (i, 1)),
        out_shape=jax.ShapeDtypeStruct((H * W, 2 * D2), jnp.float32),
        input_output_aliases={1: 0},
    )(col_embed, sc_out)
    return out


# speedup vs baseline: 1.0361x; 1.0361x over previous
import functools

import jax
import jax.numpy as jnp
from jax import lax
from jax.experimental import pallas as pl
from jax.experimental.pallas import tpu as pltpu
from jax.experimental.pallas import tpu_sc as plsc


def kernel(row_embed, col_embed):
    H, D2 = row_embed.shape
    W = col_embed.shape[0]

    NW = 32
    RPW = H // NW
    L = 16
    NV = D2 // L
    BR = 8

    mesh = plsc.VectorSubcoreMesh(core_axis_name="c", subcore_axis_name="s")

    @functools.partial(
        pl.kernel,
        mesh=mesh,
        out_type=jax.ShapeDtypeStruct((H * W, 2 * D2), jnp.float32),
        scratch_types=[
            pltpu.VMEM((RPW, D2), jnp.float32),
            pltpu.VMEM((BR, D2), jnp.float32),
            pltpu.VMEM((BR, D2), jnp.float32),
            pltpu.SemaphoreType.DMA,
        ],
    )
    def row_half_kernel(row_hbm, out_hbm, myrows_v, b0, b1, wsem):
        wid = lax.axis_index("s") * 2 + lax.axis_index("c")
        base_i = wid * RPW
        pltpu.sync_copy(row_hbm.at[pl.ds(base_i, RPW)], myrows_v)
        writes = []
        bufs = (b0, b1)
        for t in range(RPW):
            bcast = bufs[t]
            vals = [myrows_v[t, pl.ds(v * L, L)] for v in range(NV)]
            for r in range(BR):
                for v in range(NV):
                    bcast[r, pl.ds(v * L, L)] = vals[v]
            row0 = (base_i + t) * W
            for q in range(W // BR):
                writes.append(
                    pltpu.async_copy(
                        bcast,
                        out_hbm.at[pl.ds(row0 + q * BR, BR), pl.ds(0, D2)],
                        wsem,
                    )
                )
        for wcp in writes:
            wcp.wait()

    sc_out = row_half_kernel(row_embed)

    BLK = 32

    def col_half_body(col_ref, _, out_ref):
        for k in range(BLK):
            out_ref[k * W:(k + 1) * W, :] = col_ref[...]

    out = pl.pallas_call(
        col_half_body,
        grid=(H // BLK,),
        in_specs=[
            pl.BlockSpec((W, D2), lambda i: (0, 0)),
            pl.BlockSpec(memory_space=pl.ANY),
        ],
        out_specs=pl.BlockSpec((BLK * W, D2), lambda i: (i, 1)),
        out_shape=jax.ShapeDtypeStruct((H * W, 2 * D2), jnp.float32),
        input_output_aliases={1: 0},
    )(col_embed, sc_out)
    return out
